# baseline (speedup 1.0000x reference)
import functools

import jax
import jax.numpy as jnp
import numpy as np
from jax import lax
from jax.experimental import pallas as pl
from jax.experimental.pallas import tpu as pltpu
from jax.experimental.pallas import tpu_sc as plsc

_N_BINS = 15
_COLS = 4096


def _rowstats_kernel(x_ref, lab_ref, out_ref):
    x = x_ref[...]
    n_cls = x.shape[0]
    m = jnp.max(x, axis=0, keepdims=True)
    it = lax.broadcasted_iota(jnp.int32, x.shape, 0)
    first_amax = jnp.min(jnp.where(x == m, it, n_cls), axis=0)
    z = jnp.sum(jnp.exp(x - m), axis=0)
    lab = lab_ref[0, 0, :]
    out_ref[0, 0, :] = 1.0 / z
    out_ref[0, 1, :] = (first_amax == lab).astype(jnp.float32)


def _final_kernel(cp_ref, ap_ref, o_ref, *, n):
    d = jnp.abs(jnp.sum(cp_ref[...], axis=0) - jnp.sum(ap_ref[...], axis=0))
    o_ref[0, 0] = jnp.sum(d) / n


def _make_sc_binner(n):
    n_workers = 32
    chunk = n // n_workers
    mesh = plsc.VectorSubcoreMesh(core_axis_name="c", subcore_axis_name="s")
    bounds = np.linspace(0.0, 1.0, _N_BINS + 1).astype(np.float32)

    @functools.partial(
        pl.kernel,
        mesh=mesh,
        out_type=jax.ShapeDtypeStruct((2 * n_workers * 16,), jnp.float32),
        compiler_params=pltpu.CompilerParams(needs_layout_passes=False),
        scratch_types=[
            pltpu.VMEM((chunk,), jnp.float32),
            pltpu.VMEM((chunk,), jnp.float32),
            pltpu.VMEM((256,), jnp.float32),
            pltpu.VMEM((256,), jnp.float32),
            pltpu.VMEM((16,), jnp.float32),
        ],
    )
    def sc_bin(stats_hbm, out_hbm, conf_v, acc_v, hc_v, ha_v, res_v):
        wid = lax.axis_index("s") * 2 + lax.axis_index("c")
        pltpu.sync_copy(stats_hbm.at[pl.ds(2 * wid * chunk, chunk)], conf_v)
        pltpu.sync_copy(
            stats_hbm.at[pl.ds((2 * wid + 1) * chunk, chunk)], acc_v
        )
        zero16 = jnp.zeros((16,), jnp.float32)
        for r in range(16):
            hc_v[pl.ds(r * 16, 16)] = zero16
            ha_v[pl.ds(r * 16, 16)] = zero16
        lane16 = lax.iota(jnp.int32, 16) * 16

        def body(i, carry):
            for u in range(4):
                off = i * 64 + u * 16
                c = conf_v[pl.ds(off, 16)]
                a = acc_v[pl.ds(off, 16)]
                b = jnp.zeros((16,), jnp.int32)
                for t in range(1, _N_BINS):
                    b = b + jnp.where(c > bounds[t], 1, 0).astype(jnp.int32)
                flat = lane16 + b
                plsc.addupdate_scatter(hc_v, [flat], c)
                plsc.addupdate_scatter(ha_v, [flat], a)
            return carry

        lax.fori_loop(0, chunk // 64, body, 0)

        sc = zero16
        sa = zero16
        for r in range(16):
            sc = sc + hc_v[pl.ds(r * 16, 16)]
            sa = sa + ha_v[pl.ds(r * 16, 16)]
        res_v[...] = sc
        pltpu.sync_copy(res_v, out_hbm.at[pl.ds(wid * 16, 16)])
        res_v[...] = sa
        pltpu.sync_copy(res_v, out_hbm.at[pl.ds(n_workers * 16 + wid * 16, 16)])

    return sc_bin


def kernel(logits, labels):
    n, n_cls = logits.shape
    g = n // _COLS
    labels3 = labels.astype(jnp.int32).reshape(g, 1, _COLS)
    logits_t = logits.T

    stats3 = pl.pallas_call(
        _rowstats_kernel,
        grid=(g,),
        in_specs=[
            pl.BlockSpec((n_cls, _COLS), lambda i: (0, i)),
            pl.BlockSpec((1, 1, _COLS), lambda i: (i, 0, 0)),
        ],
        out_specs=pl.BlockSpec((1, 2, _COLS), lambda i: (i, 0, 0)),
        out_shape=jax.ShapeDtypeStruct((g, 2, _COLS), jnp.float32),
    )(logits_t, labels3)

    partials = _make_sc_binner(n)(stats3.reshape(2 * n))
    conf_p = partials[:512].reshape(32, 16)
    acc_p = partials[512:].reshape(32, 16)

    ece = pl.pallas_call(
        functools.partial(_final_kernel, n=n),
        out_shape=jax.ShapeDtypeStruct((1, 1), jnp.float32),
        out_specs=pl.BlockSpec(memory_space=pltpu.SMEM),
    )(conf_p, acc_p)
    return ece.reshape(1)


# speedup vs baseline: 1.0094x; 1.0094x over previous
import functools

import jax
import jax.numpy as jnp
import numpy as np
from jax import lax
from jax.experimental import pallas as pl
from jax.experimental.pallas import tpu as pltpu
from jax.experimental.pallas import tpu_sc as plsc

_N_BINS = 15
_COLS = 4096


def _rowstats_kernel(x_ref, lab_ref, conf_ref, acc_ref):
    x = x_ref[...]
    n_cls = x.shape[0]
    m = jnp.max(x, axis=0, keepdims=True)
    it = lax.broadcasted_iota(jnp.int32, x.shape, 0)
    first_amax = jnp.min(jnp.where(x == m, it, n_cls), axis=0)
    z = jnp.sum(jnp.exp(x - m), axis=0)
    lab = lab_ref[0, 0, :]
    conf_ref[0, 0, :] = 1.0 / z
    acc_ref[0, 0, :] = (first_amax == lab).astype(jnp.float32)


def _final_kernel(cp_ref, ap_ref, o_ref, *, n):
    d = jnp.abs(jnp.sum(cp_ref[...], axis=0) - jnp.sum(ap_ref[...], axis=0))
    o_ref[0, 0] = jnp.sum(d) / n


def _make_sc_binner(n):
    n_workers = 32
    chunk = n // n_workers
    mesh = plsc.VectorSubcoreMesh(core_axis_name="c", subcore_axis_name="s")
    bounds = np.linspace(0.0, 1.0, _N_BINS + 1).astype(np.float32)

    @functools.partial(
        pl.kernel,
        mesh=mesh,
        out_type=jax.ShapeDtypeStruct((2 * n_workers * 16,), jnp.float32),
        compiler_params=pltpu.CompilerParams(needs_layout_passes=False),
        scratch_types=[
            pltpu.VMEM((chunk,), jnp.float32),
            pltpu.VMEM((chunk,), jnp.float32),
            pltpu.VMEM((256,), jnp.float32),
            pltpu.VMEM((256,), jnp.float32),
            pltpu.VMEM((16,), jnp.float32),
        ],
    )
    def sc_bin(conf_hbm, acc_hbm, out_hbm, conf_v, acc_v, hc_v, ha_v, res_v):
        wid = lax.axis_index("s") * 2 + lax.axis_index("c")
        base = wid * chunk
        pltpu.sync_copy(conf_hbm.at[pl.ds(base, chunk)], conf_v)
        pltpu.sync_copy(acc_hbm.at[pl.ds(base, chunk)], acc_v)
        zero16 = jnp.zeros((16,), jnp.float32)
        for r in range(16):
            hc_v[pl.ds(r * 16, 16)] = zero16
            ha_v[pl.ds(r * 16, 16)] = zero16
        lane16 = lax.iota(jnp.int32, 16) * 16

        def body(i, carry):
            for u in range(4):
                off = i * 64 + u * 16
                c = conf_v[pl.ds(off, 16)]
                a = acc_v[pl.ds(off, 16)]
                b = jnp.zeros((16,), jnp.int32)
                for t in range(1, _N_BINS):
                    b = b + jnp.where(c > bounds[t], 1, 0).astype(jnp.int32)
                flat = lane16 + b
                plsc.addupdate_scatter(hc_v, [flat], c)
                plsc.addupdate_scatter(ha_v, [flat], a)
            return carry

        lax.fori_loop(0, chunk // 64, body, 0)

        sc = zero16
        sa = zero16
        for r in range(16):
            sc = sc + hc_v[pl.ds(r * 16, 16)]
            sa = sa + ha_v[pl.ds(r * 16, 16)]
        res_v[...] = sc
        pltpu.sync_copy(res_v, out_hbm.at[pl.ds(wid * 16, 16)])
        res_v[...] = sa
        pltpu.sync_copy(res_v, out_hbm.at[pl.ds(n_workers * 16 + wid * 16, 16)])

    return sc_bin


def kernel(logits, labels):
    n, n_cls = logits.shape
    g = n // _COLS
    labels3 = labels.astype(jnp.int32).reshape(g, 1, _COLS)
    logits_t = logits.T

    conf3, acc3 = pl.pallas_call(
        _rowstats_kernel,
        grid=(g,),
        in_specs=[
            pl.BlockSpec((n_cls, _COLS), lambda i: (0, i)),
            pl.BlockSpec((1, 1, _COLS), lambda i: (i, 0, 0)),
        ],
        out_specs=[
            pl.BlockSpec((1, 1, _COLS), lambda i: (i, 0, 0)),
            pl.BlockSpec((1, 1, _COLS), lambda i: (i, 0, 0)),
        ],
        out_shape=[
            jax.ShapeDtypeStruct((g, 1, _COLS), jnp.float32),
            jax.ShapeDtypeStruct((g, 1, _COLS), jnp.float32),
        ],
    )(logits_t, labels3)

    partials = _make_sc_binner(n)(conf3.reshape(n), acc3.reshape(n))
    conf_p = partials[:512].reshape(32, 16)
    acc_p = partials[512:].reshape(32, 16)

    ece = pl.pallas_call(
        functools.partial(_final_kernel, n=n),
        out_shape=jax.ShapeDtypeStruct((1, 1), jnp.float32),
        out_specs=pl.BlockSpec(memory_space=pltpu.SMEM),
    )(conf_p, acc_p)
    return ece.reshape(1)
